# baseline (speedup 1.0000x reference)
import functools

import jax
import jax.numpy as jnp
from jax import lax
from jax.experimental import pallas as pl
from jax.experimental.pallas import tpu as pltpu
from jax.experimental.pallas import tpu_sc as plsc

N_NODES = 10000
N_EDGES = 320000
D = 128

NC = 2
NS = 16
CHUNK = 128

NCHUNKS = 2560
E_PAD = NCHUNKS * CHUNK
CORE_CH = NCHUNKS // NC
W_CH = CORE_CH // NS

STRIPE = 640
NPAD = NS * STRIPE

RI = 8
LI = 6

_mesh = plsc.VectorSubcoreMesh(core_axis_name="c", subcore_axis_name="s")


def _pad_edges(src, dst):
    pad = E_PAD - N_EDGES
    ar = jnp.arange(pad, dtype=jnp.int32)
    src_p = jnp.concatenate([src, ar % N_NODES])
    dst_p = jnp.concatenate([dst, N_NODES + ar % (NPAD - N_NODES)])
    return src_p, dst_p


def _sc_degree(dst, zeros_d, ones_d):
    R = 8
    T = W_CH // R
    K = 6

    @functools.partial(
        pl.kernel,
        out_type=jax.ShapeDtypeStruct((NC, NPAD, D), jnp.float32),
        mesh=_mesh,
        scratch_types=(
            [pltpu.VMEM((CHUNK,), jnp.int32) for _ in range(R)]
            + [pltpu.VMEM((CHUNK, D), jnp.float32),
               pltpu.VMEM_SHARED((NPAD, D), jnp.float32)]
            + [pltpu.SemaphoreType.DMA for _ in range(2 * R)]
        ),
    )
    def deg_kernel(dst_hbm, z_hbm, one_hbm, out_hbm, *scratch):
        dstv = scratch[:R]
        ones = scratch[R]
        acc = scratch[R + 1]
        isem = scratch[R + 2:R + 2 + R]
        ssem = scratch[R + 2 + R:R + 2 + 2 * R]

        cid = lax.axis_index("c")
        sid = lax.axis_index("s")
        pltpu.sync_copy(one_hbm, ones)
        pltpu.sync_copy(z_hbm.at[pl.ds(sid * STRIPE, STRIPE)],
                        acc.at[pl.ds(sid * STRIPE, STRIPE)])
        plsc.subcore_barrier()

        wbase = (cid * CORE_CH + sid * W_CH) * CHUNK

        def start_idx(c, b):
            pltpu.async_copy(dst_hbm.at[pl.ds(wbase + c * CHUNK, CHUNK)],
                             dstv[b], isem[b])

        def wait_idx(b):
            pltpu.make_async_copy(dst_hbm.at[pl.ds(0, CHUNK)], dstv[b],
                                  isem[b]).wait()

        def start_scat(b):
            pltpu.async_copy(ones, acc.at[dstv[b]], ssem[b], add=True)

        def wait_scat(b):
            pltpu.make_async_copy(ones, acc.at[dstv[b]], ssem[b]).wait()

        for b in range(K):
            start_idx(b, b)

        @pl.loop(0, T)
        def _(t):
            c0 = t * R
            for b in range(R):
                wait_idx(b)
                start_scat(b)
                bk = (b + R - K) % R
                if b < K:
                    @pl.when(t > 0)
                    def _():
                        wait_scat(bk)
                    start_idx(c0 + b + R - K, bk)
                else:
                    wait_scat(bk)

                    @pl.when(t < T - 1)
                    def _():
                        start_idx(c0 + b + R - K, bk)

        for b in range(K):
            wait_scat((W_CH - K + b) % R)

        plsc.subcore_barrier()
        pltpu.sync_copy(acc.at[pl.ds(sid * STRIPE, STRIPE)],
                        out_hbm.at[cid, pl.ds(sid * STRIPE, STRIPE)])

    return deg_kernel(dst, zeros_d, ones_d)


GCH = 64
RR = 4
SW_CH = (E_PAD // GCH) // (NC * NS)
SCORE_CH = NS * SW_CH


def _sc_segsum(y, src, dst, zeros_d):
    T = SW_CH // RI

    @functools.partial(
        pl.kernel,
        out_type=jax.ShapeDtypeStruct((NC, NPAD, D), jnp.float32),
        mesh=_mesh,
        scratch_types=(
            [pltpu.VMEM((GCH,), jnp.int32) for _ in range(2 * RI)]
            + [pltpu.VMEM((GCH, D), jnp.float32) for _ in range(RR)]
            + [pltpu.VMEM_SHARED((NPAD, D), jnp.float32)]
            + [pltpu.SemaphoreType.DMA for _ in range(RI + 2 * RR)]
        ),
    )
    def seg_kernel(y_hbm, src_hbm, dst_hbm, z_hbm, out_hbm, *scratch):
        srcv = scratch[:RI]
        dstv = scratch[RI:2 * RI]
        rows = scratch[2 * RI:2 * RI + RR]
        acc = scratch[2 * RI + RR]
        base = 2 * RI + RR + 1
        isem = scratch[base:base + RI]
        gsem = scratch[base + RI:base + RI + RR]
        ssem = scratch[base + RI + RR:base + RI + 2 * RR]

        cid = lax.axis_index("c")
        sid = lax.axis_index("s")
        pltpu.sync_copy(z_hbm.at[pl.ds(sid * STRIPE, STRIPE)],
                        acc.at[pl.ds(sid * STRIPE, STRIPE)])
        plsc.subcore_barrier()

        wbase = (cid * SCORE_CH + sid * SW_CH) * GCH

        def start_idx(c, bi):
            pltpu.async_copy(src_hbm.at[pl.ds(wbase + c * GCH, GCH)],
                             srcv[bi], isem[bi])
            pltpu.async_copy(dst_hbm.at[pl.ds(wbase + c * GCH, GCH)],
                             dstv[bi], isem[bi])

        def wait_idx(bi):
            pltpu.make_async_copy(src_hbm.at[pl.ds(0, GCH)], srcv[bi],
                                  isem[bi]).wait()
            pltpu.make_async_copy(dst_hbm.at[pl.ds(0, GCH)], dstv[bi],
                                  isem[bi]).wait()

        def start_gath(bi, br):
            pltpu.async_copy(y_hbm.at[srcv[bi]], rows[br], gsem[br])

        def wait_gath(bi, br):
            pltpu.make_async_copy(y_hbm.at[srcv[bi]], rows[br],
                                  gsem[br]).wait()

        def start_scat(bi, br):
            pltpu.async_copy(rows[br], acc.at[dstv[bi]], ssem[br], add=True)

        def wait_scat(bi, br):
            pltpu.make_async_copy(rows[br], acc.at[dstv[bi]],
                                  ssem[br]).wait()

        for b in range(6):
            start_idx(b, b)
        wait_idx(0)
        start_gath(0, 0)
        wait_idx(1)
        start_gath(1, 1)

        @pl.loop(0, T)
        def _(t):
            c0 = t * RI
            for b in range(RI):
                br = b % RR
                br2 = (b + 2) % RR
                bi2 = (b + 2) % RI
                bi6 = (b + 6) % RI
                wait_gath(b, br)
                start_scat(b, br)
                if b < 2:
                    @pl.when(t > 0)
                    def _():
                        wait_scat(bi6, br2)
                    start_idx(c0 + b + 6, bi6)
                else:
                    wait_scat(bi6, br2)

                    @pl.when(t < T - 1)
                    def _():
                        start_idx(c0 + b + 6, bi6)
                if b < RI - 2:
                    wait_idx(bi2)
                    start_gath(bi2, br2)
                else:
                    @pl.when(t < T - 1)
                    def _():
                        wait_idx(bi2)
                        start_gath(bi2, br2)

        wait_scat((SW_CH - 2) % RI, (SW_CH - 2) % RR)
        wait_scat((SW_CH - 1) % RI, (SW_CH - 1) % RR)

        plsc.subcore_barrier()
        pltpu.sync_copy(acc.at[pl.ds(sid * STRIPE, STRIPE)],
                        out_hbm.at[cid, pl.ds(sid * STRIPE, STRIPE)])

    return seg_kernel(y, src, dst, zeros_d)


def _tc_prescale(deg2, x, W1):
    def body(d_ref, x_ref, w_ref, dinv_ref, y_ref):
        deg = d_ref[0] + d_ref[1]
        dinv = lax.rsqrt(jnp.maximum(deg, 1.0))
        dinv_ref[...] = dinv
        z1 = jnp.dot(x_ref[...], w_ref[...],
                     preferred_element_type=jnp.float32)
        y_ref[...] = z1 * dinv[:N_NODES, 0:1]

    return pl.pallas_call(
        body,
        out_shape=(
            jax.ShapeDtypeStruct((NPAD, D), jnp.float32),
            jax.ShapeDtypeStruct((N_NODES, D), jnp.float32),
        ),
    )(deg2, x, W1)


def _tc_mid(s1, dinv2d, W2, b1):
    def body(s_ref, dinv_ref, w_ref, b_ref, y2_ref):
        dinv = dinv_ref[:N_NODES, 0:1]
        agg = (s_ref[0, :N_NODES] + s_ref[1, :N_NODES]) * dinv
        h = jnp.maximum(agg + b_ref[...], 0.0)
        z2 = jnp.dot(h, w_ref[...], preferred_element_type=jnp.float32)
        y2_ref[...] = z2 * dinv

    return pl.pallas_call(
        body,
        out_shape=jax.ShapeDtypeStruct((N_NODES, D), jnp.float32),
    )(s1, dinv2d, W2, b1)


def _tc_out(s2, dinv2d, b2):
    def body(s_ref, dinv_ref, b_ref, o_ref):
        dinv = dinv_ref[:N_NODES, 0:1]
        o_ref[...] = ((s_ref[0, :N_NODES] + s_ref[1, :N_NODES]) * dinv
                      + b_ref[...])

    return pl.pallas_call(
        body,
        out_shape=jax.ShapeDtypeStruct((N_NODES, D), jnp.float32),
    )(s2, dinv2d, b2)


def kernel(features, edge_index, W1, b1, W2, b2):
    src = edge_index[0].astype(jnp.int32)
    dst = edge_index[1].astype(jnp.int32)
    src_p, dst_p = _pad_edges(src, dst)
    zeros_d = jnp.zeros((NPAD, D), jnp.float32)
    ones_d = jnp.ones((CHUNK, D), jnp.float32)

    deg2 = _sc_degree(dst_p, zeros_d, ones_d)
    dinv2d, y1 = _tc_prescale(deg2, features, W1)
    s1 = _sc_segsum(y1, src_p, dst_p, zeros_d)
    y2 = _tc_mid(s1, dinv2d, W2, b1)
    s2 = _sc_segsum(y2, src_p, dst_p, zeros_d)
    return _tc_out(s2, dinv2d, b2)


# speedup vs baseline: 1.0024x; 1.0024x over previous
import functools

import jax
import jax.numpy as jnp
from jax import lax
from jax.experimental import pallas as pl
from jax.experimental.pallas import tpu as pltpu
from jax.experimental.pallas import tpu_sc as plsc

N_NODES = 10000
N_EDGES = 320000
D = 128

NC = 2
NS = 16
CHUNK = 128

NCHUNKS = 2560
E_PAD = NCHUNKS * CHUNK
CORE_CH = NCHUNKS // NC
W_CH = CORE_CH // NS

STRIPE = 640
NPAD = NS * STRIPE

RI = 8
LI = 6

_mesh = plsc.VectorSubcoreMesh(core_axis_name="c", subcore_axis_name="s")


def _pad_edges(src, dst):
    pad = E_PAD - N_EDGES
    ar = jnp.arange(pad, dtype=jnp.int32)
    src_p = jnp.concatenate([src, ar % N_NODES])
    dst_p = jnp.concatenate([dst, N_NODES + ar % (NPAD - N_NODES)])
    return src_p, dst_p


def _sc_degree(dst, zeros_d, ones_d):
    R = 8
    T = W_CH // R
    K = 4

    @functools.partial(
        pl.kernel,
        out_type=jax.ShapeDtypeStruct((NC, NPAD, D), jnp.float32),
        mesh=_mesh,
        scratch_types=(
            [pltpu.VMEM((CHUNK,), jnp.int32) for _ in range(R)]
            + [pltpu.VMEM((CHUNK, D), jnp.float32),
               pltpu.VMEM_SHARED((NPAD, D), jnp.float32)]
            + [pltpu.SemaphoreType.DMA for _ in range(2 * R)]
        ),
    )
    def deg_kernel(dst_hbm, z_hbm, one_hbm, out_hbm, *scratch):
        dstv = scratch[:R]
        ones = scratch[R]
        acc = scratch[R + 1]
        isem = scratch[R + 2:R + 2 + R]
        ssem = scratch[R + 2 + R:R + 2 + 2 * R]

        cid = lax.axis_index("c")
        sid = lax.axis_index("s")
        pltpu.sync_copy(one_hbm, ones)
        pltpu.sync_copy(z_hbm.at[pl.ds(sid * STRIPE, STRIPE)],
                        acc.at[pl.ds(sid * STRIPE, STRIPE)])
        plsc.subcore_barrier()

        wbase = (cid * CORE_CH + sid * W_CH) * CHUNK

        def start_idx(c, b):
            pltpu.async_copy(dst_hbm.at[pl.ds(wbase + c * CHUNK, CHUNK)],
                             dstv[b], isem[b])

        def wait_idx(b):
            pltpu.make_async_copy(dst_hbm.at[pl.ds(0, CHUNK)], dstv[b],
                                  isem[b]).wait()

        def start_scat(b):
            pltpu.async_copy(ones, acc.at[dstv[b]], ssem[b], add=True)

        def wait_scat(b):
            pltpu.make_async_copy(ones, acc.at[dstv[b]], ssem[b]).wait()

        for b in range(K):
            start_idx(b, b)

        @pl.loop(0, T)
        def _(t):
            c0 = t * R
            for b in range(R):
                wait_idx(b)
                start_scat(b)
                bk = (b + R - K) % R
                if b < K:
                    @pl.when(t > 0)
                    def _():
                        wait_scat(bk)
                    start_idx(c0 + b + R - K, bk)
                else:
                    wait_scat(bk)

                    @pl.when(t < T - 1)
                    def _():
                        start_idx(c0 + b + R - K, bk)

        for b in range(K):
            wait_scat((W_CH - K + b) % R)

        plsc.subcore_barrier()
        pltpu.sync_copy(acc.at[pl.ds(sid * STRIPE, STRIPE)],
                        out_hbm.at[cid, pl.ds(sid * STRIPE, STRIPE)])

    return deg_kernel(dst, zeros_d, ones_d)


GCH = 64
RR = 4
SW_CH = (E_PAD // GCH) // (NC * NS)
SCORE_CH = NS * SW_CH


def _sc_segsum(y, src, dst, zeros_d):
    T = SW_CH // RI

    @functools.partial(
        pl.kernel,
        out_type=jax.ShapeDtypeStruct((NC, NPAD, D), jnp.float32),
        mesh=_mesh,
        scratch_types=(
            [pltpu.VMEM((GCH,), jnp.int32) for _ in range(2 * RI)]
            + [pltpu.VMEM((GCH, D), jnp.float32) for _ in range(RR)]
            + [pltpu.VMEM_SHARED((NPAD, D), jnp.float32)]
            + [pltpu.SemaphoreType.DMA for _ in range(RI + 2 * RR)]
        ),
    )
    def seg_kernel(y_hbm, src_hbm, dst_hbm, z_hbm, out_hbm, *scratch):
        srcv = scratch[:RI]
        dstv = scratch[RI:2 * RI]
        rows = scratch[2 * RI:2 * RI + RR]
        acc = scratch[2 * RI + RR]
        base = 2 * RI + RR + 1
        isem = scratch[base:base + RI]
        gsem = scratch[base + RI:base + RI + RR]
        ssem = scratch[base + RI + RR:base + RI + 2 * RR]

        cid = lax.axis_index("c")
        sid = lax.axis_index("s")
        pltpu.sync_copy(z_hbm.at[pl.ds(sid * STRIPE, STRIPE)],
                        acc.at[pl.ds(sid * STRIPE, STRIPE)])
        plsc.subcore_barrier()

        wbase = (cid * SCORE_CH + sid * SW_CH) * GCH

        def start_idx(c, bi):
            pltpu.async_copy(src_hbm.at[pl.ds(wbase + c * GCH, GCH)],
                             srcv[bi], isem[bi])
            pltpu.async_copy(dst_hbm.at[pl.ds(wbase + c * GCH, GCH)],
                             dstv[bi], isem[bi])

        def wait_idx(bi):
            pltpu.make_async_copy(src_hbm.at[pl.ds(0, GCH)], srcv[bi],
                                  isem[bi]).wait()
            pltpu.make_async_copy(dst_hbm.at[pl.ds(0, GCH)], dstv[bi],
                                  isem[bi]).wait()

        def start_gath(bi, br):
            pltpu.async_copy(y_hbm.at[srcv[bi]], rows[br], gsem[br])

        def wait_gath(bi, br):
            pltpu.make_async_copy(y_hbm.at[srcv[bi]], rows[br],
                                  gsem[br]).wait()

        def start_scat(bi, br):
            pltpu.async_copy(rows[br], acc.at[dstv[bi]], ssem[br], add=True)

        def wait_scat(bi, br):
            pltpu.make_async_copy(rows[br], acc.at[dstv[bi]],
                                  ssem[br]).wait()

        for b in range(6):
            start_idx(b, b)
        wait_idx(0)
        start_gath(0, 0)
        wait_idx(1)
        start_gath(1, 1)

        @pl.loop(0, T)
        def _(t):
            c0 = t * RI
            for b in range(RI):
                br = b % RR
                br2 = (b + 2) % RR
                bi2 = (b + 2) % RI
                bi6 = (b + 6) % RI
                wait_gath(b, br)
                start_scat(b, br)
                if b < 2:
                    @pl.when(t > 0)
                    def _():
                        wait_scat(bi6, br2)
                    start_idx(c0 + b + 6, bi6)
                else:
                    wait_scat(bi6, br2)

                    @pl.when(t < T - 1)
                    def _():
                        start_idx(c0 + b + 6, bi6)
                if b < RI - 2:
                    wait_idx(bi2)
                    start_gath(bi2, br2)
                else:
                    @pl.when(t < T - 1)
                    def _():
                        wait_idx(bi2)
                        start_gath(bi2, br2)

        wait_scat((SW_CH - 2) % RI, (SW_CH - 2) % RR)
        wait_scat((SW_CH - 1) % RI, (SW_CH - 1) % RR)

        plsc.subcore_barrier()
        pltpu.sync_copy(acc.at[pl.ds(sid * STRIPE, STRIPE)],
                        out_hbm.at[cid, pl.ds(sid * STRIPE, STRIPE)])

    return seg_kernel(y, src, dst, zeros_d)


def _tc_prescale(deg2, x, W1):
    def body(d_ref, x_ref, w_ref, dinv_ref, y_ref):
        deg = d_ref[0] + d_ref[1]
        dinv = lax.rsqrt(jnp.maximum(deg, 1.0))
        dinv_ref[...] = dinv
        z1 = jnp.dot(x_ref[...], w_ref[...],
                     preferred_element_type=jnp.float32)
        y_ref[...] = z1 * dinv[:N_NODES, 0:1]

    return pl.pallas_call(
        body,
        out_shape=(
            jax.ShapeDtypeStruct((NPAD, D), jnp.float32),
            jax.ShapeDtypeStruct((N_NODES, D), jnp.float32),
        ),
    )(deg2, x, W1)


def _tc_mid(s1, dinv2d, W2, b1):
    def body(s_ref, dinv_ref, w_ref, b_ref, y2_ref):
        dinv = dinv_ref[:N_NODES, 0:1]
        agg = (s_ref[0, :N_NODES] + s_ref[1, :N_NODES]) * dinv
        h = jnp.maximum(agg + b_ref[...], 0.0)
        z2 = jnp.dot(h, w_ref[...], preferred_element_type=jnp.float32)
        y2_ref[...] = z2 * dinv

    return pl.pallas_call(
        body,
        out_shape=jax.ShapeDtypeStruct((N_NODES, D), jnp.float32),
    )(s1, dinv2d, W2, b1)


def _tc_out(s2, dinv2d, b2):
    def body(s_ref, dinv_ref, b_ref, o_ref):
        dinv = dinv_ref[:N_NODES, 0:1]
        o_ref[...] = ((s_ref[0, :N_NODES] + s_ref[1, :N_NODES]) * dinv
                      + b_ref[...])

    return pl.pallas_call(
        body,
        out_shape=jax.ShapeDtypeStruct((N_NODES, D), jnp.float32),
    )(s2, dinv2d, b2)


def kernel(features, edge_index, W1, b1, W2, b2):
    src = edge_index[0].astype(jnp.int32)
    dst = edge_index[1].astype(jnp.int32)
    src_p, dst_p = _pad_edges(src, dst)
    zeros_d = jnp.zeros((NPAD, D), jnp.float32)
    ones_d = jnp.ones((CHUNK, D), jnp.float32)

    deg2 = _sc_degree(dst_p, zeros_d, ones_d)
    dinv2d, y1 = _tc_prescale(deg2, features, W1)
    s1 = _sc_segsum(y1, src_p, dst_p, zeros_d)
    y2 = _tc_mid(s1, dinv2d, W2, b1)
    s2 = _sc_segsum(y2, src_p, dst_p, zeros_d)
    return _tc_out(s2, dinv2d, b2)
